# baseline (speedup 1.0000x reference)
import functools

import jax
import jax.numpy as jnp
from jax import lax
from jax.experimental import pallas as pl
from jax.experimental.pallas import tpu as pltpu
from jax.experimental.pallas import tpu_sc as plsc

_CHUNK_ROWS = 256


def kernel(seq_len, table):
    del seq_len
    rows, dim = table.shape
    info = plsc.get_sparse_core_info()
    rows_per_core = rows // info.num_cores
    nchunk = rows_per_core // _CHUNK_ROWS

    mesh = plsc.ScalarSubcoreMesh(axis_name="c", num_cores=info.num_cores)

    @functools.partial(
        pl.kernel,
        mesh=mesh,
        out_type=jax.ShapeDtypeStruct((rows, dim), table.dtype),
        scratch_types=[
            pltpu.VMEM_SHARED((_CHUNK_ROWS, dim), jnp.float32),
            pltpu.VMEM_SHARED((_CHUNK_ROWS, dim), jnp.float32),
            pltpu.SemaphoreType.DMA,
            pltpu.SemaphoreType.DMA,
            pltpu.SemaphoreType.DMA,
            pltpu.SemaphoreType.DMA,
        ],
    )
    def scs_copy(table_hbm, out_hbm, buf0, buf1, li0, li1, so0, so1):
        base = lax.axis_index("c") * rows_per_core
        bufs = (buf0, buf1)
        load_sems = (li0, li1)
        store_sems = (so0, so1)

        def start_load(c, b):
            return pltpu.async_copy(
                table_hbm.at[pl.ds(base + c * _CHUNK_ROWS, _CHUNK_ROWS)],
                bufs[b],
                load_sems[b],
            )

        def start_store(c, b):
            return pltpu.async_copy(
                bufs[b],
                out_hbm.at[pl.ds(base + c * _CHUNK_ROWS, _CHUNK_ROWS)],
                store_sems[b],
            )

        loads = [None, None]
        stores = [None, None]
        loads[0] = start_load(0, 0)
        for c in range(nchunk):
            b = c & 1
            nb = (c + 1) & 1
            if c + 1 < nchunk:
                if stores[nb] is not None:
                    stores[nb].wait()
                loads[nb] = start_load(c + 1, nb)
            loads[b].wait()
            stores[b] = start_store(c, b)
        for b in range(2):
            if stores[b] is not None:
                stores[b].wait()

    out = scs_copy(table)
    return out[None]
